# Initial kernel scaffold; baseline (speedup 1.0000x reference)
import jax
import jax.numpy as jnp
from jax.experimental import pallas as pl


def kernel(x, edge_index, edge_weight, batch, W1, W2, Wp, bp, gamma, beta, alpha):
    raise NotImplementedError("write your pallas kernel here")


# speedup vs baseline: 5.2939x; 5.2939x over previous
import functools

import jax
import jax.numpy as jnp
from jax import lax
from jax.experimental import pallas as pl
from jax.experimental.pallas import tpu as pltpu
from jax.experimental.pallas import tpu_sc as plsc

N, E, D, G = 10000, 320000, 128, 512
NP = 10240
NC, NS, L = 2, 16, 16
NW = NC * NS
ECH = 128
NCHUNKS = E // ECH
RCH = 16
NRCH = NP // RCH
NXCH = N // RCH
BLK = 1024
NBLK = NP // BLK


def _sc_edge_agg(x, src, dst, ew):
    mesh = plsc.VectorSubcoreMesh(core_axis_name="c", subcore_axis_name="s",
                                  num_cores=NC, num_subcores=NS)

    @functools.partial(
        pl.kernel,
        out_type=jax.ShapeDtypeStruct((NC, NP, D), jnp.float32),
        mesh=mesh,
        scratch_types=[
            pltpu.VMEM((RCH, D), jnp.float32),
            pltpu.VMEM((RCH, D), jnp.float32),
            pltpu.VMEM((ECH,), jnp.int32),
            pltpu.VMEM((ECH,), jnp.int32),
            pltpu.VMEM((ECH,), jnp.float32),
            pltpu.VMEM((ECH, D), jnp.float32),
            pltpu.VMEM_SHARED((NP, D), jnp.float32),
            pltpu.SemaphoreType.DMA,
        ],
    )
    def k(x_hbm, src_hbm, dst_hbm, ew_hbm, p_hbm,
          tmp_v, zrow_v, src_v, dst_v, ew_v, rows_v, agg_sh, sem):
        cid = lax.axis_index("c")
        sid = lax.axis_index("s")
        zero = jnp.zeros((L,), jnp.float32)
        for r in range(RCH):
            for cb in range(D // L):
                zrow_v[r, pl.ds(cb * L, L)] = zero

        def init_body(kk, carry):
            rc = kk * NS + sid
            r0 = rc * RCH
            is_x = jnp.logical_and(cid == 0, rc < NXCH)

            @pl.when(is_x)
            def _():
                pltpu.sync_copy(x_hbm.at[pl.ds(r0, RCH)], tmp_v)
                pltpu.sync_copy(tmp_v, agg_sh.at[pl.ds(r0, RCH)])

            @pl.when(jnp.logical_not(is_x))
            def _():
                pltpu.sync_copy(zrow_v, agg_sh.at[pl.ds(r0, RCH)])

            return carry

        lax.fori_loop(0, NRCH // NS, init_body, 0)
        plsc.subcore_barrier()

        wid = sid * NC + cid

        def edge_body(kk, carry):
            chunk = kk * NW + wid

            @pl.when(chunk < NCHUNKS)
            def _():
                e0 = chunk * ECH
                pltpu.sync_copy(src_hbm.at[pl.ds(e0, ECH)], src_v)
                pltpu.sync_copy(dst_hbm.at[pl.ds(e0, ECH)], dst_v)
                pltpu.sync_copy(ew_hbm.at[pl.ds(e0, ECH)], ew_v)
                pltpu.async_copy(x_hbm.at[src_v], rows_v, sem).wait()

                dnums = lax.GatherDimensionNumbers(
                    offset_dims=(), collapsed_slice_dims=(0,),
                    start_index_map=(0,))

                def grp_body(gg, c2):
                    r0 = gg * L
                    ewv = ew_v[pl.ds(r0, L)]
                    for j in range(L):
                        w = lax.gather(
                            ewv, jnp.full((L, 1), j, jnp.int32), dnums, (1,),
                            mode=lax.GatherScatterMode.PROMISE_IN_BOUNDS)
                        for cb in range(D // L):
                            sl = pl.ds(cb * L, L)
                            rows_v[r0 + j, sl] = rows_v[r0 + j, sl] * w
                    return c2

                lax.fori_loop(0, ECH // L, grp_body, 0)
                pltpu.sync_copy(rows_v, agg_sh.at[dst_v], add=True)

            return carry

        lax.fori_loop(0, (NCHUNKS + NW - 1) // NW, edge_body, 0)
        plsc.subcore_barrier()

        def wb_body(kk, carry):
            r0 = (kk * NS + sid) * RCH
            pltpu.sync_copy(agg_sh.at[pl.ds(r0, RCH)], tmp_v)
            pltpu.sync_copy(tmp_v, p_hbm.at[cid, pl.ds(r0, RCH)])
            return carry

        lax.fori_loop(0, NRCH // NS, wb_body, 0)

    return k(x, src, dst, ew)


def _tc_post(p, batch3, W1, W2, Wp, bp2, gamma2, beta2, alpha2):
    def body(p_ref, b_ref, W1_ref, W2_ref, Wp_ref, bp_ref, gam_ref, bet_ref,
             al_ref, g_ref, gt_ref, hp_ref, sagg):
        i = pl.program_id(0)
        agg = p_ref[0] + p_ref[1]
        h_on = jnp.dot(agg, W2_ref[...], preferred_element_type=jnp.float32)
        z = jnp.dot(h_on, Wp_ref[...], preferred_element_type=jnp.float32)
        z = z + bp_ref[...]
        mu = jnp.mean(z, axis=-1, keepdims=True)
        var = jnp.mean((z - mu) ** 2, axis=-1, keepdims=True)
        z = (z - mu) / jnp.sqrt(var + 1e-5) * gam_ref[...] + bet_ref[...]
        alpha = al_ref[0, 0]
        hp_ref[...] = jnp.where(z >= 0, z, alpha * z)

        bvec = b_ref[0, 0, :]
        seg = lax.broadcasted_iota(jnp.int32, (G, BLK), 0)
        mask = (bvec[None, :] == seg).astype(jnp.float32)
        part = jnp.dot(mask, agg, preferred_element_type=jnp.float32)

        @pl.when(i == 0)
        def _():
            sagg[...] = part

        @pl.when(i > 0)
        def _():
            sagg[...] = sagg[...] + part

        @pl.when(i == NBLK - 1)
        def _():
            s = sagg[...]
            g_ref[...] = jnp.dot(s, W1_ref[...], preferred_element_type=jnp.float32)
            gt_ref[...] = jnp.dot(s, W2_ref[...], preferred_element_type=jnp.float32)

    return pl.pallas_call(
        body,
        grid=(NBLK,),
        in_specs=[
            pl.BlockSpec((NC, BLK, D), lambda i: (0, i, 0)),
            pl.BlockSpec((1, 1, BLK), lambda i: (i, 0, 0)),
            pl.BlockSpec((D, D), lambda i: (0, 0)),
            pl.BlockSpec((D, D), lambda i: (0, 0)),
            pl.BlockSpec((D, D), lambda i: (0, 0)),
            pl.BlockSpec((1, D), lambda i: (0, 0)),
            pl.BlockSpec((1, D), lambda i: (0, 0)),
            pl.BlockSpec((1, D), lambda i: (0, 0)),
            pl.BlockSpec((1, 1), lambda i: (0, 0)),
        ],
        out_specs=[
            pl.BlockSpec((G, D), lambda i: (0, 0)),
            pl.BlockSpec((G, D), lambda i: (0, 0)),
            pl.BlockSpec((BLK, D), lambda i: (i, 0)),
        ],
        out_shape=[
            jax.ShapeDtypeStruct((G, D), jnp.float32),
            jax.ShapeDtypeStruct((G, D), jnp.float32),
            jax.ShapeDtypeStruct((NP, D), jnp.float32),
        ],
        scratch_shapes=[pltpu.VMEM((G, D), jnp.float32)],
    )(p, batch3, W1, W2, Wp, bp2, gamma2, beta2, alpha2)


def kernel(x, edge_index, edge_weight, batch, W1, W2, Wp, bp, gamma, beta, alpha):
    src = edge_index[0]
    dst = edge_index[1]
    p = _sc_edge_agg(x, src, dst, edge_weight)
    batch_pad = jnp.concatenate([batch, jnp.full((NP - N,), G, jnp.int32)])
    batch3 = batch_pad.reshape(NBLK, 1, BLK)
    g, gt, hp = _tc_post(
        p, batch3, W1, W2, Wp,
        bp.reshape(1, D), gamma.reshape(1, D), beta.reshape(1, D),
        jnp.asarray(alpha, jnp.float32).reshape(1, 1),
    )
    h_pred = hp[:N]
    return (g, g, h_pred, h_pred, gt, gt)
